# baseline (speedup 1.0000x reference)
import functools

import jax
import jax.numpy as jnp
from jax import lax
from jax.experimental import pallas as pl
from jax.experimental.pallas import tpu as pltpu
from jax.experimental.pallas import tpu_sc as plsc

_B = 8
_NB = 32768
_NA = 8192
_K = 16
_DB = 64
_DA = 64
_H = 128

_G = 8
_BPG = _B // _G

_NC, _NS = 2, 16
_NW = _NC * _NS
_APW = (_BPG * _NA) // _NW
_RPG = 128
_APG = _RPG // _K
_STEPS = (_APW * _K) // _RPG
_FLUSH = 16
_NBUF = 2


def _sc_body(table, idxm, out, idx_v, buf0, buf1, out_v, sem0, sem1):
    w = lax.axis_index("s") * _NC + lax.axis_index("c")
    pltpu.sync_copy(idxm.at[pl.ds(w * _STEPS, _STEPS)], idx_v)

    bufs = (buf0, buf1)
    sems = (sem0, sem1)
    for p in range(_NBUF):
        pltpu.async_copy(table.at[idx_v.at[p]], bufs[p], sems[p])
    wpb = _NA // _APW
    out_b = w // wpb
    out_row = (w % wpb) * _APW

    def reduce_step(s, buf):
        row0 = (s % _FLUSH) * _APG
        for k in range(_APG):
            for d in range(_DB // 32):
                acc = buf[k * _K, d]
                for j in range(1, _K):
                    acc = acc + buf[k * _K + j, d]
                out_v[row0 + k, d] = acc

    def body(t, carry):
        for p in range(_NBUF):
            s = _NBUF * t + p
            buf, sem = bufs[p], sems[p]
            pltpu.make_async_copy(table.at[idx_v.at[0]], buf, sem).wait()
            reduce_step(s, buf)
            nxt = jnp.minimum(s + _NBUF, _STEPS - 1)
            pltpu.async_copy(table.at[idx_v.at[nxt]], buf, sem)

        @pl.when((t + 1) % (_FLUSH // _NBUF) == 0)
        def _flush():
            blk = (_NBUF * t + _NBUF - 1) // _FLUSH
            dst = out.at[out_b].at[
                pl.ds(out_row + blk * (_FLUSH * _APG), _FLUSH * _APG)]
            pltpu.sync_copy(out_v, dst)

        return carry

    lax.fori_loop(0, _STEPS // _NBUF, body, None)
    for p in range(_NBUF):
        pltpu.make_async_copy(table.at[idx_v.at[0]], bufs[p], sems[p]).wait()


@functools.lru_cache(maxsize=1)
def _sc_gather_sum():
    return pl.kernel(
        _sc_body,
        out_type=jax.ShapeDtypeStruct((_BPG, _NA, _DB // 32, 32),
                                      jnp.bfloat16),
        mesh=plsc.VectorSubcoreMesh(
            core_axis_name="c", subcore_axis_name="s",
            num_cores=_NC, num_subcores=_NS,
        ),
        scratch_types=[
            pltpu.VMEM((_STEPS, _RPG), jnp.int32),
            pltpu.VMEM((_RPG, _DB // 32, 32), jnp.bfloat16),
            pltpu.VMEM((_RPG, _DB // 32, 32), jnp.bfloat16),
            pltpu.VMEM((_FLUSH * _APG, _DB // 32, 32), jnp.bfloat16),
            pltpu.SemaphoreType.DMA,
            pltpu.SemaphoreType.DMA,
        ],
        compiler_params=pltpu.CompilerParams(use_tc_tiling_on_sc=False),
    )


def _tc_body(af_ref, msg_ref, w1_ref, w2_ref, b_ref, out_ref):
    af = af_ref[0]
    msg = msg_ref[0]
    h = jnp.dot(af, w1_ref[...], preferred_element_type=jnp.float32)
    h = h + jnp.dot(msg, w2_ref[...], preferred_element_type=jnp.float32)
    h = jnp.maximum(h + b_ref[...], 0.0)
    i = pl.program_id(0)
    out_ref[pl.ds(i, 1), :] = jnp.sum(h, axis=0, keepdims=True)


def _tc_readout(af, msg, w1, w2, b):
    return pl.pallas_call(
        _tc_body,
        grid=(_BPG,),
        in_specs=[
            pl.BlockSpec((1, _NA, _DA), lambda i: (i, 0, 0)),
            pl.BlockSpec((1, _NA, _DB), lambda i: (i, 0, 0)),
            pl.BlockSpec((_DA, _H), lambda i: (0, 0)),
            pl.BlockSpec((_DB, _H), lambda i: (0, 0)),
            pl.BlockSpec((1, _H), lambda i: (0, 0)),
        ],
        out_specs=pl.BlockSpec((_BPG, _H), lambda i: (0, 0)),
        out_shape=jax.ShapeDtypeStruct((_BPG, _H), jnp.float32),
    )(af, msg, w1, w2, b)


def kernel(bond_representations, atomic_features, atom_bond_neighbors, W_o, b_o):
    br = bond_representations.reshape(_B, _NB, _DB)
    offs = (jnp.arange(_BPG, dtype=jnp.int32) * _NB)[:, None, None]
    w1 = W_o[:_DA]
    w2 = W_o[_DA:]
    b = b_o.reshape(1, _H)
    parts = []
    for g in range(_G):
        sl = slice(g * _BPG, (g + 1) * _BPG)
        table = br[sl].reshape(_BPG * _NB, _DB // 32, 32).astype(jnp.bfloat16)
        idx = (atom_bond_neighbors[sl].astype(jnp.int32) + offs).reshape(-1, _RPG)
        msg = _sc_gather_sum()(table, idx).reshape(_BPG, _NA, _DB)
        parts.append(_tc_readout(atomic_features[sl], msg, w1, w2, b))
    return jnp.concatenate(parts, axis=0)


# speedup vs baseline: 2.0990x; 2.0990x over previous
import functools

import jax
import jax.numpy as jnp
from jax import lax
from jax.experimental import pallas as pl
from jax.experimental.pallas import tpu as pltpu
from jax.experimental.pallas import tpu_sc as plsc

_B = 8
_NB = 32768
_NA = 8192
_K = 16
_DB = 64
_DA = 64
_H = 128

_G = 4
_BPG = _B // _G

_NC, _NS = 2, 16
_NW = _NC * _NS
_APW = (_BPG * _NA) // _NW
_RPG = 128
_APG = _RPG // _K
_STEPS = (_APW * _K) // _RPG
_FLUSH = 16
_NBUF = 2


def _sc_body(table, idxm, out, idx_v, buf0, buf1, out_v, sem0, sem1):
    w = lax.axis_index("s") * _NC + lax.axis_index("c")
    pltpu.sync_copy(idxm.at[pl.ds(w * _STEPS, _STEPS)], idx_v)

    bufs = (buf0, buf1)
    sems = (sem0, sem1)
    for p in range(_NBUF):
        pltpu.async_copy(table.at[idx_v.at[p]], bufs[p], sems[p])
    wpb = _NA // _APW
    out_b = w // wpb
    out_row = (w % wpb) * _APW

    def reduce_step(s, buf):
        row0 = (s % _FLUSH) * _APG
        for k in range(_APG):
            for d in range(_DB // 32):
                acc = buf[k * _K, d]
                for j in range(1, _K):
                    acc = acc + buf[k * _K + j, d]
                out_v[row0 + k, d] = acc

    def body(t, carry):
        for p in range(_NBUF):
            s = _NBUF * t + p
            buf, sem = bufs[p], sems[p]
            pltpu.make_async_copy(table.at[idx_v.at[0]], buf, sem).wait()
            reduce_step(s, buf)
            nxt = jnp.minimum(s + _NBUF, _STEPS - 1)
            pltpu.async_copy(table.at[idx_v.at[nxt]], buf, sem)

        @pl.when((t + 1) % (_FLUSH // _NBUF) == 0)
        def _flush():
            blk = (_NBUF * t + _NBUF - 1) // _FLUSH
            dst = out.at[out_b].at[
                pl.ds(out_row + blk * (_FLUSH * _APG), _FLUSH * _APG)]
            pltpu.sync_copy(out_v, dst)

        return carry

    lax.fori_loop(0, _STEPS // _NBUF, body, None)
    for p in range(_NBUF):
        pltpu.make_async_copy(table.at[idx_v.at[0]], bufs[p], sems[p]).wait()


@functools.lru_cache(maxsize=1)
def _sc_gather_sum():
    return pl.kernel(
        _sc_body,
        out_type=jax.ShapeDtypeStruct((_BPG, _NA, _DB // 32, 32),
                                      jnp.bfloat16),
        mesh=plsc.VectorSubcoreMesh(
            core_axis_name="c", subcore_axis_name="s",
            num_cores=_NC, num_subcores=_NS,
        ),
        scratch_types=[
            pltpu.VMEM((_STEPS, _RPG), jnp.int32),
            pltpu.VMEM((_RPG, _DB // 32, 32), jnp.bfloat16),
            pltpu.VMEM((_RPG, _DB // 32, 32), jnp.bfloat16),
            pltpu.VMEM((_FLUSH * _APG, _DB // 32, 32), jnp.bfloat16),
            pltpu.SemaphoreType.DMA,
            pltpu.SemaphoreType.DMA,
        ],
        compiler_params=pltpu.CompilerParams(use_tc_tiling_on_sc=False),
    )


def _tc_body(af_ref, msg_ref, w1_ref, w2_ref, b_ref, out_ref):
    af = af_ref[0]
    msg = msg_ref[0]
    h = jnp.dot(af, w1_ref[...], preferred_element_type=jnp.float32)
    h = h + jnp.dot(msg, w2_ref[...], preferred_element_type=jnp.float32)
    h = jnp.maximum(h + b_ref[...], 0.0)
    i = pl.program_id(0)
    out_ref[pl.ds(i, 1), :] = jnp.sum(h, axis=0, keepdims=True)


def _tc_readout(af, msg, w1, w2, b):
    return pl.pallas_call(
        _tc_body,
        grid=(_BPG,),
        in_specs=[
            pl.BlockSpec((1, _NA, _DA), lambda i: (i, 0, 0)),
            pl.BlockSpec((1, _NA, _DB), lambda i: (i, 0, 0)),
            pl.BlockSpec((_DA, _H), lambda i: (0, 0)),
            pl.BlockSpec((_DB, _H), lambda i: (0, 0)),
            pl.BlockSpec((1, _H), lambda i: (0, 0)),
        ],
        out_specs=pl.BlockSpec((_BPG, _H), lambda i: (0, 0)),
        out_shape=jax.ShapeDtypeStruct((_BPG, _H), jnp.float32),
    )(af, msg, w1, w2, b)


def kernel(bond_representations, atomic_features, atom_bond_neighbors, W_o, b_o):
    br = bond_representations.reshape(_B, _NB, _DB)
    offs = (jnp.arange(_BPG, dtype=jnp.int32) * _NB)[:, None, None]
    w1 = W_o[:_DA]
    w2 = W_o[_DA:]
    b = b_o.reshape(1, _H)
    parts = []
    for g in range(_G):
        sl = slice(g * _BPG, (g + 1) * _BPG)
        table = br[sl].reshape(_BPG * _NB, _DB // 32, 32).astype(jnp.bfloat16)
        idx = (atom_bond_neighbors[sl].astype(jnp.int32) + offs).reshape(-1, _RPG)
        msg = _sc_gather_sum()(table, idx).reshape(_BPG, _NA, _DB)
        parts.append(_tc_readout(atomic_features[sl], msg, w1, w2, b))
    return jnp.concatenate(parts, axis=0)
